# baseline (speedup 1.0000x reference)
import functools

import jax
import jax.numpy as jnp
from jax import lax
from jax.experimental import pallas as pl
from jax.experimental.pallas import tpu as pltpu
from jax.experimental.pallas import tpu_sc as plsc

DIM = 64
B = 4096
L = 50
NB = B * L
NC = 2
NS = 16
NW = NC * NS
NSC = NB - 4096
BPW = NSC // NW
CW = 392
NCHUNK = BPW // CW
NBUF = 4

_mesh = plsc.VectorSubcoreMesh(core_axis_name="c", subcore_axis_name="s")


@functools.partial(
    pl.kernel,
    mesh=_mesh,
    out_type=jax.ShapeDtypeStruct((NSC, DIM), jnp.float32),
    scratch_types=[
        pltpu.VMEM((BPW,), jnp.int32),
        pltpu.VMEM((NBUF, CW, DIM), jnp.float32),
        pltpu.SemaphoreType.DMA,
        pltpu.SemaphoreType.DMA,
        pltpu.SemaphoreType.DMA,
    ],
    compiler_params=pltpu.CompilerParams(use_tc_tiling_on_sc=False),
)
def _embed_gather(idx_hbm, table_hbm, out_hbm, idx_v, rows_v, gsem, osem, isem):
    wid = lax.axis_index("s") * NC + lax.axis_index("c")
    base = wid * BPW

    head = (NBUF - 1) * CW
    pltpu.sync_copy(idx_hbm.at[pl.ds(base, head)], idx_v.at[pl.ds(0, head)])
    for b in range(NBUF - 1):
        pltpu.async_copy(
            table_hbm.at[idx_v.at[pl.ds(b * CW, CW)]], rows_v.at[b], gsem
        )
    pltpu.async_copy(
        idx_hbm.at[pl.ds(base + head, BPW - head)],
        idx_v.at[pl.ds(head, BPW - head)],
        isem,
    ).wait()

    def body(i, carry):
        for b in range(NBUF):
            j = i * NBUF + b
            pltpu.make_async_copy(
                table_hbm.at[idx_v.at[pl.ds(0, CW)]], rows_v.at[b], gsem
            ).wait()
            pltpu.async_copy(
                rows_v.at[b], out_hbm.at[pl.ds(base + j * CW, CW)], osem
            )
            @pl.when(j > 0)
            def _():
                pltpu.make_async_copy(
                    rows_v.at[(b - 1) % NBUF],
                    out_hbm.at[pl.ds(base, CW)],
                    osem,
                ).wait()

            nxt = j + NBUF - 1

            @pl.when(nxt < NCHUNK)
            def _():
                pltpu.async_copy(
                    table_hbm.at[idx_v.at[pl.ds(nxt * CW, CW)]],
                    rows_v.at[(b - 1) % NBUF],
                    gsem,
                )

        return carry

    lax.fori_loop(0, NCHUNK // NBUF, body, 0)
    pltpu.make_async_copy(
        rows_v.at[(NCHUNK - 1) % NBUF], out_hbm.at[pl.ds(base, CW)], osem
    ).wait()


NTC = 4096
RB = 512
NSTEP = NTC // RB


def _tc_body(idx_s, table_hbm, out_ref, *sems):
    nq = len(sems)

    def issue(g, c):
        for k in range(nq):
            r = g * nq + k
            pltpu.make_async_copy(
                table_hbm.at[pl.ds(idx_s[0, 0, r], 1)],
                out_ref.at[pl.ds(r, 1)],
                sems[k],
            ).start()
        return c

    lax.fori_loop(0, RB // nq, issue, 0)
    for k in range(nq):
        pltpu.make_async_copy(
            table_hbm.at[pl.ds(0, RB // nq)],
            out_ref.at[pl.ds(k * (RB // nq), RB // nq)],
            sems[k],
        ).wait()


def kernel(x, weight):
    xf = x.reshape(NB).astype(jnp.int32)
    idx_tc = xf[:NTC].reshape(NSTEP, 1, RB)
    idx_sc = xf[NTC:]

    out_tc = pl.pallas_call(
        _tc_body,
        grid=(NSTEP,),
        in_specs=[
            pl.BlockSpec((1, 1, RB), lambda i: (i, 0, 0), memory_space=pltpu.SMEM),
            pl.BlockSpec(memory_space=pl.ANY),
        ],
        out_specs=pl.BlockSpec((RB, DIM), lambda i: (i, 0)),
        out_shape=jax.ShapeDtypeStruct((NTC, DIM), jnp.float32),
        scratch_shapes=[pltpu.SemaphoreType.DMA] * 8,
    )(idx_tc, weight)

    out_sc = _embed_gather(idx_sc, weight)
    out = jnp.concatenate([out_tc, out_sc], axis=0)
    return out.reshape(B, L, DIM)


# speedup vs baseline: 1.3770x; 1.3770x over previous
import functools

import jax
import jax.numpy as jnp
from jax import lax
from jax.experimental import pallas as pl
from jax.experimental.pallas import tpu as pltpu
from jax.experimental.pallas import tpu_sc as plsc

DIM = 64
B = 4096
L = 50
NB = B * L
NC = 2
NS = 16
NW = NC * NS
BPW = NB // NW
CW = 200
NCHUNK = BPW // CW
NBUF = 8

_mesh = plsc.VectorSubcoreMesh(core_axis_name="c", subcore_axis_name="s")


@functools.partial(
    pl.kernel,
    mesh=_mesh,
    out_type=jax.ShapeDtypeStruct((NB, DIM), jnp.float32),
    scratch_types=[
        pltpu.VMEM((BPW,), jnp.int32),
        pltpu.VMEM((NBUF, CW, DIM), jnp.float32),
        pltpu.SemaphoreType.DMA,
        pltpu.SemaphoreType.DMA,
        pltpu.SemaphoreType.DMA,
    ],
    compiler_params=pltpu.CompilerParams(use_tc_tiling_on_sc=False),
)
def _embed_gather(idx_hbm, table_hbm, out_hbm, idx_v, rows_v, gsem, osem, isem):
    wid = lax.axis_index("s") * NC + lax.axis_index("c")
    base = wid * BPW

    head = (NBUF - 1) * CW
    pltpu.sync_copy(idx_hbm.at[pl.ds(base, head)], idx_v.at[pl.ds(0, head)])
    for b in range(NBUF - 1):
        pltpu.async_copy(
            table_hbm.at[idx_v.at[pl.ds(b * CW, CW)]], rows_v.at[b], gsem
        )
    pltpu.async_copy(
        idx_hbm.at[pl.ds(base + head, BPW - head)],
        idx_v.at[pl.ds(head, BPW - head)],
        isem,
    ).wait()

    def body(i, carry):
        for b in range(NBUF):
            j = i * NBUF + b
            pltpu.make_async_copy(
                table_hbm.at[idx_v.at[pl.ds(0, CW)]], rows_v.at[b], gsem
            ).wait()
            pltpu.async_copy(
                rows_v.at[b], out_hbm.at[pl.ds(base + j * CW, CW)], osem
            )
            @pl.when(j > 0)
            def _():
                pltpu.make_async_copy(
                    rows_v.at[(b - 1) % NBUF],
                    out_hbm.at[pl.ds(base, CW)],
                    osem,
                ).wait()

            nxt = j + NBUF - 1

            @pl.when(nxt < NCHUNK)
            def _():
                pltpu.async_copy(
                    table_hbm.at[idx_v.at[pl.ds(nxt * CW, CW)]],
                    rows_v.at[(b - 1) % NBUF],
                    gsem,
                )

        return carry

    lax.fori_loop(0, NCHUNK // NBUF, body, 0)
    pltpu.make_async_copy(
        rows_v.at[(NCHUNK - 1) % NBUF], out_hbm.at[pl.ds(base, CW)], osem
    ).wait()


def kernel(x, weight):
    idx = x.reshape(NB).astype(jnp.int32)
    out = _embed_gather(idx, weight)
    return out.reshape(B, L, DIM)


# speedup vs baseline: 1.3793x; 1.0016x over previous
import functools

import jax
import jax.numpy as jnp
from jax import lax
from jax.experimental import pallas as pl
from jax.experimental.pallas import tpu as pltpu
from jax.experimental.pallas import tpu_sc as plsc

DIM = 64
B = 4096
L = 50
NB = B * L
NC = 2
NS = 16
NW = NC * NS
BPW = NB // NW
CW = 400
NCHUNK = BPW // CW
NBUF = 4

_mesh = plsc.VectorSubcoreMesh(core_axis_name="c", subcore_axis_name="s")


@functools.partial(
    pl.kernel,
    mesh=_mesh,
    out_type=jax.ShapeDtypeStruct((NB, DIM), jnp.float32),
    scratch_types=[
        pltpu.VMEM((BPW,), jnp.int32),
        pltpu.VMEM((NBUF, CW, DIM), jnp.float32),
        pltpu.SemaphoreType.DMA,
        pltpu.SemaphoreType.DMA,
        pltpu.SemaphoreType.DMA,
    ],
    compiler_params=pltpu.CompilerParams(use_tc_tiling_on_sc=False),
)
def _embed_gather(idx_hbm, table_hbm, out_hbm, idx_v, rows_v, gsem, osem, isem):
    wid = lax.axis_index("s") * NC + lax.axis_index("c")
    base = wid * BPW

    head = (NBUF - 1) * CW
    pltpu.sync_copy(idx_hbm.at[pl.ds(base, head)], idx_v.at[pl.ds(0, head)])
    for b in range(NBUF - 1):
        pltpu.async_copy(
            table_hbm.at[idx_v.at[pl.ds(b * CW, CW)]], rows_v.at[b], gsem
        )
    pltpu.async_copy(
        idx_hbm.at[pl.ds(base + head, BPW - head)],
        idx_v.at[pl.ds(head, BPW - head)],
        isem,
    ).wait()

    def body(i, carry):
        for b in range(NBUF):
            j = i * NBUF + b
            pltpu.make_async_copy(
                table_hbm.at[idx_v.at[pl.ds(0, CW)]], rows_v.at[b], gsem
            ).wait()
            pltpu.async_copy(
                rows_v.at[b], out_hbm.at[pl.ds(base + j * CW, CW)], osem
            )
            @pl.when(j > 0)
            def _():
                pltpu.make_async_copy(
                    rows_v.at[(b - 1) % NBUF],
                    out_hbm.at[pl.ds(base, CW)],
                    osem,
                ).wait()

            nxt = j + NBUF - 1

            @pl.when(nxt < NCHUNK)
            def _():
                pltpu.async_copy(
                    table_hbm.at[idx_v.at[pl.ds(nxt * CW, CW)]],
                    rows_v.at[(b - 1) % NBUF],
                    gsem,
                )

        return carry

    lax.fori_loop(0, NCHUNK // NBUF, body, 0)
    pltpu.make_async_copy(
        rows_v.at[(NCHUNK - 1) % NBUF], out_hbm.at[pl.ds(base, CW)], osem
    ).wait()


def kernel(x, weight):
    idx = x.reshape(NB).astype(jnp.int32)
    out = _embed_gather(idx, weight)
    return out.reshape(B, L, DIM)
